# Initial kernel scaffold; baseline (speedup 1.0000x reference)
import jax
import jax.numpy as jnp
from jax.experimental import pallas as pl


def kernel(sequence, token_table):
    raise NotImplementedError("write your pallas kernel here")


# speedup vs baseline: 3.1666x; 3.1666x over previous
import functools

import jax
import jax.numpy as jnp
import numpy as np
from jax import lax
from jax.experimental import pallas as pl
from jax.experimental.pallas import tpu as pltpu
from jax.experimental.pallas import tpu_sc as plsc

B = 1024
L = 200
D = 128
NC = 2
NS = 16
NW = NC * NS
ROWS_PER_W = B * L // NW
CH = 128
NCH = ROWS_PER_W // CH
LANES = 16


def _pos_encoding():
    pos = np.arange(L, dtype=np.float64)[:, None]
    i = np.arange(D // 2, dtype=np.float64)[None, :]
    angles = pos / np.power(10000.0, (2.0 * i) / D)
    pe = np.zeros((L, D), dtype=np.float32)
    pe[:, 0::2] = np.sin(angles)
    pe[:, 1::2] = np.cos(angles)
    return pe


_PE = _pos_encoding()


@functools.partial(
    pl.kernel,
    out_type=jax.ShapeDtypeStruct((NW, NCH, CH, D), jnp.float32),
    mesh=plsc.VectorSubcoreMesh(core_axis_name="c", subcore_axis_name="s"),
    scratch_types=[
        pltpu.VMEM((NCH, CH), jnp.int32),
        pltpu.VMEM((L, D), jnp.float32),
        pltpu.VMEM((CH, D), jnp.float32),
        pltpu.VMEM((CH, D), jnp.float32),
        pltpu.SemaphoreType.DMA,
        pltpu.SemaphoreType.DMA,
    ],
)
def _emb_lookup(seq_hbm, pe_hbm, table_hbm, out_hbm,
                idx_v, pe_v, buf0, buf1, sem0, sem1):
    w = lax.axis_index("s") * NC + lax.axis_index("c")

    pltpu.sync_copy(seq_hbm.at[w], idx_v)
    pltpu.sync_copy(pe_hbm, pe_v)

    pltpu.async_copy(table_hbm.at[idx_v.at[0]], buf0, sem0)
    pltpu.async_copy(table_hbm.at[idx_v.at[1]], buf1, sem1)

    def do_chunk(c, buf, sem):
        pltpu.make_async_copy(table_hbm.at[idx_v.at[c]], buf, sem).wait()
        l0 = lax.rem(c * CH, L)

        def row(i, carry):
            l = l0 + i
            l = jnp.where(l >= L, l - L, l)
            for j in range(D // LANES):
                sl = pl.ds(j * LANES, LANES)
                plsc.addupdate(buf.at[i, sl], pe_v[l, sl])
            return carry

        lax.fori_loop(0, CH, row, 0, unroll=2)
        pltpu.sync_copy(buf, out_hbm.at[w, c])

        nxt = c + 2

        @pl.when(nxt < NCH)
        def _():
            pltpu.async_copy(table_hbm.at[idx_v.at[nxt]], buf, sem)

    def outer(c2, carry):
        do_chunk(c2 * 2, buf0, sem0)
        do_chunk(c2 * 2 + 1, buf1, sem1)
        return carry

    lax.fori_loop(0, NCH // 2, outer, 0)


def kernel(sequence, token_table):
    seq3 = sequence.reshape(NW, NCH, CH)
    out = _emb_lookup(seq3, jnp.asarray(_PE), token_table)
    return out.reshape(B, L, D)
